# baseline (speedup 1.0000x reference)
import functools

import jax
import jax.numpy as jnp
from jax import lax
from jax.experimental import pallas as pl
from jax.experimental.pallas import tpu as pltpu
from jax.experimental.pallas import tpu_sc as plsc

UNITS = 4
LB = -17.0
UB = 8.0
STEP = 0.05
NUM_BUCKETS = int((UB - LB) / STEP) + 1
RESIDUE = LB - STEP

_NB_PAD = 512
_U_PAD = 8
_B = 16384
_TOTAL = _B * UNITS
_XC = 128
_XR = _TOTAL // _XC
_NW = 32
_WROWS = _XR // _NW


def _prep_body(v_ref, b_ref, w_ref, q_ref):
    v = v_ref[...]
    w = jnp.maximum(v, 0.0)
    row = lax.broadcasted_iota(jnp.int32, (_NB_PAD, _NB_PAD), 0)
    col = lax.broadcasted_iota(jnp.int32, (_NB_PAD, _NB_PAD), 1)
    m = jnp.where(row < col, jnp.float32(1.0), jnp.float32(0.0))
    p = jax.lax.dot(w, m, precision=jax.lax.Precision.HIGHEST)
    w_ref[...] = w
    q_ref[...] = p * jnp.float32(STEP) + jnp.float32(RESIDUE) + b_ref[...]


def _prep_tables(v_pad, b_pad):
    return pl.pallas_call(
        _prep_body,
        out_shape=[
            jax.ShapeDtypeStruct((_U_PAD, _NB_PAD), jnp.float32),
            jax.ShapeDtypeStruct((_U_PAD, _NB_PAD), jnp.float32),
        ],
    )(v_pad, b_pad)


def _sc_body(x_hbm, q_hbm, w_hbm, out_hbm, x_v, q_v, w_v, o_v):
    wid = lax.axis_index("s") * 2 + lax.axis_index("c")
    base = wid * _WROWS
    pltpu.sync_copy(x_hbm.at[pl.ds(base, _WROWS), :], x_v)
    pltpu.sync_copy(q_hbm, q_v)
    pltpu.sync_copy(w_hbm, w_v)

    lane = lax.iota(jnp.int32, 16)
    u_off = lax.shift_left(lax.bitwise_and(lane, 3), 9)

    c_lb = jnp.float32(LB + 1e-09)
    c_ub = jnp.float32(UB - 1e-09)
    c_lbs = jnp.float32(LB)
    c_step = jnp.float32(STEP)

    @plsc.parallel_loop(0, _WROWS * 8, unroll=8)
    def _loop(i):
        r = lax.shift_right_logical(i, 3)
        off = lax.shift_left(lax.bitwise_and(i, 7), 4)
        xv = x_v[r, pl.ds(off, 16)]
        xc = jnp.minimum(jnp.maximum(xv, c_lb), c_ub)
        t = (xc - c_lbs + c_step) / c_step
        k = t.astype(jnp.int32)
        k = jnp.minimum(jnp.maximum(k, 0), NUM_BUCKETS - 1)
        delta = xc - c_lbs + c_step - k.astype(jnp.float32) * c_step
        g = u_off + k
        qv = plsc.load_gather(q_v, [g])
        wv = plsc.load_gather(w_v, [g])
        z = qv + delta * wv
        o_v[r, pl.ds(off, 16)] = jnp.float32(1.0) / (
            jnp.float32(1.0) + jnp.exp(-z)
        )

    pltpu.sync_copy(o_v, out_hbm.at[pl.ds(base, _WROWS), :])


@jax.jit
def _sc_main(x2, q_flat, w_flat):
    mesh = plsc.VectorSubcoreMesh(core_axis_name="c", subcore_axis_name="s")
    f = pl.kernel(
        _sc_body,
        mesh=mesh,
        compiler_params=pltpu.CompilerParams(needs_layout_passes=False),
        out_type=jax.ShapeDtypeStruct((_XR, _XC), jnp.float32),
        scratch_types=[
            pltpu.VMEM((_WROWS, _XC), jnp.float32),
            pltpu.VMEM((_U_PAD * _NB_PAD,), jnp.float32),
            pltpu.VMEM((_U_PAD * _NB_PAD,), jnp.float32),
            pltpu.VMEM((_WROWS, _XC), jnp.float32),
        ],
    )
    return f(x2, q_flat, w_flat)


def kernel(x, v, b):
    if x.ndim == 1:
        x = jnp.broadcast_to(x[:, None], (x.shape[0], UNITS))
    v_pad = jnp.zeros((_U_PAD, _NB_PAD), jnp.float32).at[:UNITS, :NUM_BUCKETS].set(v)
    b_pad = jnp.zeros((_U_PAD, 1), jnp.float32).at[:UNITS, 0].set(b)
    w_tab, q_tab = _prep_tables(v_pad, b_pad)
    out2 = _sc_main(
        x.reshape(_XR, _XC),
        q_tab.reshape(_U_PAD * _NB_PAD),
        w_tab.reshape(_U_PAD * _NB_PAD),
    )
    return out2.reshape(_B, UNITS)


# speedup vs baseline: 1.2919x; 1.2919x over previous
import functools

import jax
import jax.numpy as jnp
from jax import lax
from jax.experimental import pallas as pl
from jax.experimental.pallas import tpu as pltpu
from jax.experimental.pallas import tpu_sc as plsc

UNITS = 4
LB = -17.0
UB = 8.0
STEP = 0.05
NUM_BUCKETS = int((UB - LB) / STEP) + 1
RESIDUE = LB - STEP

_NB_PAD = 512
_U_PAD = 8
_B = 16384
_NW = 32
_ROWS = _B // _NW
_CROWS = 256
_NCHUNK = _ROWS // _CROWS
_CVECS = _CROWS * UNITS // 16


def _prep_body(v_ref, b_ref, w_ref, q_ref):
    v = v_ref[...]
    w = jnp.maximum(v, 0.0)
    row = lax.broadcasted_iota(jnp.int32, (_NB_PAD, _NB_PAD), 0)
    col = lax.broadcasted_iota(jnp.int32, (_NB_PAD, _NB_PAD), 1)
    m = jnp.where(row < col, jnp.float32(1.0), jnp.float32(0.0))
    p = jax.lax.dot(w, m, precision=jax.lax.Precision.HIGHEST)
    w_ref[...] = w
    q_ref[...] = p * jnp.float32(STEP) + jnp.float32(RESIDUE) + b_ref[...]


def _prep_tables(v_pad, b_pad):
    return pl.pallas_call(
        _prep_body,
        out_shape=[
            jax.ShapeDtypeStruct((_U_PAD, _NB_PAD), jnp.float32),
            jax.ShapeDtypeStruct((_U_PAD, _NB_PAD), jnp.float32),
        ],
    )(v_pad, b_pad)


def _sc_body(x_hbm, q_hbm, w_hbm, out_hbm, x_v, q_v, w_v, o_v):
    wid = lax.axis_index("s") * 2 + lax.axis_index("c")
    base = wid * _ROWS
    pltpu.sync_copy(q_hbm, q_v)
    pltpu.sync_copy(w_hbm, w_v)

    lane = lax.iota(jnp.int32, 16)
    u_vec = lax.bitwise_and(lane, 3)
    r_vec = lax.shift_right_logical(lane, 2)

    c_lb = jnp.float32(LB + 1e-09)
    c_ub = jnp.float32(UB - 1e-09)
    c_lbs = jnp.float32(LB)
    c_step = jnp.float32(STEP)

    def chunk(c, _):
        crow = base + c * _CROWS
        pltpu.sync_copy(x_hbm.at[pl.ds(crow, _CROWS), :], x_v)

        @plsc.parallel_loop(0, _CVECS, unroll=8)
        def _loop(i):
            rows = r_vec + i * 4
            xv = plsc.load_gather(x_v, [rows, u_vec])
            xc = jnp.minimum(jnp.maximum(xv, c_lb), c_ub)
            t = (xc - c_lbs + c_step) / c_step
            k = t.astype(jnp.int32)
            k = jnp.minimum(jnp.maximum(k, 0), NUM_BUCKETS - 1)
            delta = xc - c_lbs + c_step - k.astype(jnp.float32) * c_step
            qv = plsc.load_gather(q_v, [u_vec, k])
            wv = plsc.load_gather(w_v, [u_vec, k])
            z = qv + delta * wv
            s = jnp.float32(1.0) / (jnp.float32(1.0) + jnp.exp(-z))
            plsc.store_scatter(o_v, [rows, u_vec], s)

        pltpu.sync_copy(o_v, out_hbm.at[pl.ds(crow, _CROWS), :])
        return 0

    lax.fori_loop(0, _NCHUNK, chunk, 0)


@jax.jit
def _sc_main(x, q_tab, w_tab):
    mesh = plsc.VectorSubcoreMesh(core_axis_name="c", subcore_axis_name="s")
    f = pl.kernel(
        _sc_body,
        mesh=mesh,
        compiler_params=pltpu.CompilerParams(needs_layout_passes=False),
        out_type=jax.ShapeDtypeStruct((_B, UNITS), jnp.float32),
        scratch_types=[
            pltpu.VMEM((_CROWS, UNITS), jnp.float32),
            pltpu.VMEM((_U_PAD, _NB_PAD), jnp.float32),
            pltpu.VMEM((_U_PAD, _NB_PAD), jnp.float32),
            pltpu.VMEM((_CROWS, UNITS), jnp.float32),
        ],
    )
    return f(x, q_tab, w_tab)


def kernel(x, v, b):
    if x.ndim == 1:
        x = jnp.broadcast_to(x[:, None], (x.shape[0], UNITS))
    v_pad = jnp.zeros((_U_PAD, _NB_PAD), jnp.float32).at[:UNITS, :NUM_BUCKETS].set(v)
    b_pad = jnp.zeros((_U_PAD, 1), jnp.float32).at[:UNITS, 0].set(b)
    w_tab, q_tab = _prep_tables(v_pad, b_pad)
    return _sc_main(x, q_tab, w_tab)


# speedup vs baseline: 2.2501x; 1.7417x over previous
import functools

import jax
import jax.numpy as jnp
from jax import lax
from jax.experimental import pallas as pl
from jax.experimental.pallas import tpu as pltpu
from jax.experimental.pallas import tpu_sc as plsc

UNITS = 4
LB = -17.0
UB = 8.0
STEP = 0.05
NUM_BUCKETS = int((UB - LB) / STEP) + 1
RESIDUE = LB - STEP

_NB_PAD = 512
_U_PAD = 8
_B = 16384
_NW = 32
_COLS = _B // _NW
_CVECS = _COLS // 16


def _prep_body(v_ref, b_ref, w_ref, q_ref):
    v = v_ref[...]
    w = jnp.maximum(v, 0.0)
    row = lax.broadcasted_iota(jnp.int32, (_NB_PAD, _NB_PAD), 0)
    col = lax.broadcasted_iota(jnp.int32, (_NB_PAD, _NB_PAD), 1)
    m = jnp.where(row < col, jnp.float32(1.0), jnp.float32(0.0))
    p = jax.lax.dot(w, m, precision=jax.lax.Precision.HIGHEST)
    w_ref[...] = w
    q_ref[...] = p * jnp.float32(STEP) + jnp.float32(RESIDUE) + b_ref[...]


def _prep_tables(v_pad, b_pad):
    return pl.pallas_call(
        _prep_body,
        out_shape=[
            jax.ShapeDtypeStruct((_U_PAD, _NB_PAD), jnp.float32),
            jax.ShapeDtypeStruct((_U_PAD, _NB_PAD), jnp.float32),
        ],
    )(v_pad, b_pad)


def _sc_body(xt_hbm, q_hbm, w_hbm, out_hbm, x_v, q_v, w_v, o_v):
    wid = lax.axis_index("s") * 2 + lax.axis_index("c")
    base = wid * _COLS
    pltpu.sync_copy(xt_hbm.at[:, pl.ds(base, _COLS)], x_v)
    pltpu.sync_copy(q_hbm, q_v)
    pltpu.sync_copy(w_hbm, w_v)

    c_lb = jnp.float32(LB + 1e-09)
    c_ub = jnp.float32(UB - 1e-09)
    c_lbs = jnp.float32(LB)
    c_step = jnp.float32(STEP)

    for u in range(UNITS):
        u_vec = jnp.full((16,), u, jnp.int32)

        @plsc.parallel_loop(0, _CVECS, unroll=8)
        def _loop(i, u=u, u_vec=u_vec):
            off = i * 16
            xv = x_v[u, pl.ds(off, 16)]
            xc = jnp.minimum(jnp.maximum(xv, c_lb), c_ub)
            t = (xc - c_lbs + c_step) / c_step
            k = t.astype(jnp.int32)
            k = jnp.minimum(jnp.maximum(k, 0), NUM_BUCKETS - 1)
            delta = xc - c_lbs + c_step - k.astype(jnp.float32) * c_step
            qv = plsc.load_gather(q_v, [u_vec, k])
            wv = plsc.load_gather(w_v, [u_vec, k])
            z = qv + delta * wv
            o_v[u, pl.ds(off, 16)] = jnp.float32(1.0) / (
                jnp.float32(1.0) + jnp.exp(-z)
            )

    pltpu.sync_copy(o_v, out_hbm.at[:, pl.ds(base, _COLS)])


@jax.jit
def _sc_main(xt, q_tab, w_tab):
    mesh = plsc.VectorSubcoreMesh(core_axis_name="c", subcore_axis_name="s")
    f = pl.kernel(
        _sc_body,
        mesh=mesh,
        compiler_params=pltpu.CompilerParams(needs_layout_passes=False),
        out_type=jax.ShapeDtypeStruct((UNITS, _B), jnp.float32),
        scratch_types=[
            pltpu.VMEM((UNITS, _COLS), jnp.float32),
            pltpu.VMEM((_U_PAD, _NB_PAD), jnp.float32),
            pltpu.VMEM((_U_PAD, _NB_PAD), jnp.float32),
            pltpu.VMEM((UNITS, _COLS), jnp.float32),
        ],
    )
    return f(xt, q_tab, w_tab)


def kernel(x, v, b):
    if x.ndim == 1:
        x = jnp.broadcast_to(x[:, None], (x.shape[0], UNITS))
    v_pad = jnp.zeros((_U_PAD, _NB_PAD), jnp.float32).at[:UNITS, :NUM_BUCKETS].set(v)
    b_pad = jnp.zeros((_U_PAD, 1), jnp.float32).at[:UNITS, 0].set(b)
    w_tab, q_tab = _prep_tables(v_pad, b_pad)
    out_t = _sc_main(x.T, q_tab, w_tab)
    return out_t.T


# speedup vs baseline: 2.2952x; 1.0200x over previous
import functools

import jax
import jax.numpy as jnp
from jax import lax
from jax.experimental import pallas as pl
from jax.experimental.pallas import tpu as pltpu
from jax.experimental.pallas import tpu_sc as plsc

UNITS = 4
LB = -17.0
UB = 8.0
STEP = 0.05
NUM_BUCKETS = int((UB - LB) / STEP) + 1
RESIDUE = LB - STEP

_NB_PAD = 512
_U_PAD = 8
_B = 16384
_NW = 32
_COLS = _B // _NW
_CVECS = _COLS // 16


def _prep_body(v_ref, b_ref, w_ref, q_ref):
    v = v_ref[...]
    w = jnp.maximum(v, 0.0)
    row = lax.broadcasted_iota(jnp.int32, (_NB_PAD, _NB_PAD), 0)
    col = lax.broadcasted_iota(jnp.int32, (_NB_PAD, _NB_PAD), 1)
    m = jnp.where(row < col, jnp.float32(1.0), jnp.float32(0.0))
    p = jax.lax.dot(w, m, precision=jax.lax.Precision.HIGHEST)
    w_ref[...] = w
    q_ref[...] = p * jnp.float32(STEP) + jnp.float32(RESIDUE) + b_ref[...]


def _prep_tables(v_pad, b_pad):
    return pl.pallas_call(
        _prep_body,
        out_shape=[
            jax.ShapeDtypeStruct((_U_PAD, _NB_PAD), jnp.float32),
            jax.ShapeDtypeStruct((_U_PAD, _NB_PAD), jnp.float32),
        ],
    )(v_pad, b_pad)


def _sc_body(xt_hbm, q_hbm, w_hbm, out_hbm, x_v, q_v, w_v, o_v, s0, s1, s2):
    wid = lax.axis_index("s") * 2 + lax.axis_index("c")
    base = wid * _COLS
    cx = pltpu.async_copy(xt_hbm.at[:, pl.ds(base, _COLS)], x_v, s0)
    cq = pltpu.async_copy(q_hbm, q_v, s1)
    cw = pltpu.async_copy(w_hbm, w_v, s2)
    cx.wait()
    cq.wait()
    cw.wait()

    c_lb = jnp.float32(LB + 1e-09)
    c_ub = jnp.float32(UB - 1e-09)
    c_lbs = jnp.float32(LB)
    c_step = jnp.float32(STEP)

    for u in range(UNITS):
        u_vec = jnp.full((16,), u, jnp.int32)

        @plsc.parallel_loop(0, _CVECS, unroll=8)
        def _loop(i, u=u, u_vec=u_vec):
            off = i * 16
            xv = x_v[u, pl.ds(off, 16)]
            xc = jnp.minimum(jnp.maximum(xv, c_lb), c_ub)
            t = (xc - c_lbs + c_step) / c_step
            k = t.astype(jnp.int32)
            k = jnp.minimum(jnp.maximum(k, 0), NUM_BUCKETS - 1)
            delta = xc - c_lbs + c_step - k.astype(jnp.float32) * c_step
            qv = plsc.load_gather(q_v, [u_vec, k])
            wv = plsc.load_gather(w_v, [u_vec, k])
            z = qv + delta * wv
            o_v[u, pl.ds(off, 16)] = jnp.float32(1.0) / (
                jnp.float32(1.0) + jnp.exp(-z)
            )

    pltpu.sync_copy(o_v, out_hbm.at[:, pl.ds(base, _COLS)])


@jax.jit
def _sc_main(xt, q_tab, w_tab):
    mesh = plsc.VectorSubcoreMesh(core_axis_name="c", subcore_axis_name="s")
    f = pl.kernel(
        _sc_body,
        mesh=mesh,
        compiler_params=pltpu.CompilerParams(needs_layout_passes=False),
        out_type=jax.ShapeDtypeStruct((UNITS, _B), jnp.float32),
        scratch_types=[
            pltpu.VMEM((UNITS, _COLS), jnp.float32),
            pltpu.VMEM((_U_PAD, _NB_PAD), jnp.float32),
            pltpu.VMEM((_U_PAD, _NB_PAD), jnp.float32),
            pltpu.VMEM((UNITS, _COLS), jnp.float32),
            pltpu.SemaphoreType.DMA,
            pltpu.SemaphoreType.DMA,
            pltpu.SemaphoreType.DMA,
        ],
    )
    return f(xt, q_tab, w_tab)


def kernel(x, v, b):
    if x.ndim == 1:
        x = jnp.broadcast_to(x[:, None], (x.shape[0], UNITS))
    v_pad = jnp.zeros((_U_PAD, _NB_PAD), jnp.float32).at[:UNITS, :NUM_BUCKETS].set(v)
    b_pad = jnp.zeros((_U_PAD, 1), jnp.float32).at[:UNITS, 0].set(b)
    w_tab, q_tab = _prep_tables(v_pad, b_pad)
    out_t = _sc_main(x.T, q_tab, w_tab)
    return out_t.T
